# Initial kernel scaffold; baseline (speedup 1.0000x reference)
import jax
import jax.numpy as jnp
from jax.experimental import pallas as pl


def kernel(x, fc1_w, fc1_b, fc2_w, fc2_b, fc3_w, fc3_b, fc4_w, fc4_b, emb):
    raise NotImplementedError("write your pallas kernel here")


# speedup vs baseline: 3.3842x; 3.3842x over previous
import jax
import jax.numpy as jnp
from jax.experimental import pallas as pl

B = 1024
IN = 784
H = 400
D = 256
K = 512


def _mm(a, b_t):
    return jax.lax.dot_general(a, b_t, (((1,), (1,)), ((), ())),
                               preferred_element_type=jnp.float32,
                               precision=jax.lax.Precision.HIGHEST)


def _vqvae_kernel(x_ref, fc1_w_ref, fc1_b_ref, fc2_w_ref, fc2_b_ref,
                  fc3_w_ref, fc3_b_ref, fc4_w_ref, fc4_b_ref, emb_ref,
                  xr_ref, rloss_ref, eloss_ref):
    f32 = jnp.float32
    x = x_ref[...]
    h1 = jnp.maximum(_mm(x, fc1_w_ref[...]) + fc1_b_ref[...], 0.0)
    z_e = _mm(h1, fc2_w_ref[...]) + fc2_b_ref[...]
    emb = emb_ref[...]
    g = _mm(z_e, emb)
    emb2 = emb * emb
    ones_row = jnp.ones((1, D), dtype=f32)
    emb_sq = _mm(ones_row, emb2)
    score = emb_sq - 2.0 * g
    m = jnp.min(score, axis=1, keepdims=True)
    lane = jax.lax.broadcasted_iota(jnp.int32, (B, K), 1)
    idx = jnp.min(jnp.where(score == m, lane, K), axis=1, keepdims=True)
    onehot = (lane == idx).astype(f32)
    z_q = jax.lax.dot_general(onehot, emb, (((1,), (0,)), ((), ())),
                              preferred_element_type=f32,
                              precision=jax.lax.Precision.HIGHEST)
    h3 = jnp.maximum(_mm(z_q, fc3_w_ref[...]) + fc3_b_ref[...], 0.0)
    logits = _mm(h3, fc4_w_ref[...]) + fc4_b_ref[...]
    x_reconst = jax.nn.sigmoid(logits)
    xr_ref[...] = x_reconst
    logp = jnp.maximum(jnp.log(x_reconst), -100.0)
    log1mp = jnp.maximum(jnp.log(1.0 - x_reconst), -100.0)
    rloss = -jnp.sum(x * logp + (1.0 - x) * log1mp) / (B * IN)
    rloss_ref[...] = rloss[None, None]
    dz = z_e - z_q
    eloss = jnp.sum(dz * dz) / B
    eloss_ref[...] = eloss[None, None]


def kernel(x, fc1_w, fc1_b, fc2_w, fc2_b, fc3_w, fc3_b, fc4_w, fc4_b, emb):
    out = pl.pallas_call(
        _vqvae_kernel,
        out_shape=(
            jax.ShapeDtypeStruct((B, IN), jnp.float32),
            jax.ShapeDtypeStruct((1, 1), jnp.float32),
            jax.ShapeDtypeStruct((1, 1), jnp.float32),
        ),
    )(x, fc1_w, fc1_b.reshape(1, H), fc2_w, fc2_b.reshape(1, D),
      fc3_w, fc3_b.reshape(1, H), fc4_w, fc4_b.reshape(1, IN), emb)
    x_reconst, rloss, eloss = out
    rl = rloss[0, 0]
    el = eloss[0, 0]
    return (x_reconst, rl, el, el)


# speedup vs baseline: 5.0786x; 1.5007x over previous
import jax
import jax.numpy as jnp
from jax.experimental import pallas as pl

B = 1024
IN = 784
H = 400
D = 256
K = 512


def _mmb(a, b_t):
    return jax.lax.dot_general(a.astype(jnp.bfloat16),
                               b_t.astype(jnp.bfloat16),
                               (((1,), (1,)), ((), ())),
                               preferred_element_type=jnp.float32)


def _mm_hi(a, b_t):
    return jax.lax.dot_general(a, b_t, (((1,), (1,)), ((), ())),
                               preferred_element_type=jnp.float32,
                               precision=jax.lax.Precision.HIGHEST)


def _vqvae_kernel(x_ref, fc1_w_ref, fc1_b_ref, fc2_w_ref, fc2_b_ref,
                  fc3_w_ref, fc3_b_ref, fc4_w_ref, fc4_b_ref, emb_ref,
                  xr_ref, rloss_ref, eloss_ref):
    f32 = jnp.float32
    x = x_ref[...]
    h1 = jnp.maximum(_mmb(x, fc1_w_ref[...]) + fc1_b_ref[...], 0.0)
    z_e = _mmb(h1, fc2_w_ref[...]) + fc2_b_ref[...]
    emb = emb_ref[...]
    g = _mm_hi(z_e, emb)
    emb_sq = _mm_hi(jnp.ones((1, D), f32), emb * emb)
    score = emb_sq - 2.0 * g
    m = jnp.min(score, axis=1, keepdims=True)
    lane = jax.lax.broadcasted_iota(jnp.int32, (B, K), 1)
    idx = jnp.min(jnp.where(score == m, lane, K), axis=1, keepdims=True)
    onehot = (lane == idx).astype(f32)
    z_q = jax.lax.dot_general(onehot, emb, (((1,), (0,)), ((), ())),
                              preferred_element_type=f32,
                              precision=jax.lax.Precision.HIGHEST)
    h3 = jnp.maximum(_mmb(z_q, fc3_w_ref[...]) + fc3_b_ref[...], 0.0)
    logits = _mmb(h3, fc4_w_ref[...]) + fc4_b_ref[...]
    x_reconst = jax.nn.sigmoid(logits)
    xr_ref[...] = x_reconst
    logp = jnp.maximum(jnp.log(x_reconst), -100.0)
    log1mp = jnp.maximum(jnp.log(1.0 - x_reconst), -100.0)
    rloss = -jnp.sum(x * logp + (1.0 - x) * log1mp) / (B * IN)
    rloss_ref[...] = rloss[None, None]
    dz = z_e - z_q
    eloss = jnp.sum(dz * dz) / B
    eloss_ref[...] = eloss[None, None]


def kernel(x, fc1_w, fc1_b, fc2_w, fc2_b, fc3_w, fc3_b, fc4_w, fc4_b, emb):
    out = pl.pallas_call(
        _vqvae_kernel,
        out_shape=(
            jax.ShapeDtypeStruct((B, IN), jnp.float32),
            jax.ShapeDtypeStruct((1, 1), jnp.float32),
            jax.ShapeDtypeStruct((1, 1), jnp.float32),
        ),
    )(x, fc1_w, fc1_b.reshape(1, H), fc2_w, fc2_b.reshape(1, D),
      fc3_w, fc3_b.reshape(1, H), fc4_w, fc4_b.reshape(1, IN), emb)
    x_reconst, rloss, eloss = out
    rl = rloss[0, 0]
    el = eloss[0, 0]
    return (x_reconst, rl, el, el)
